# baseline (device time: 61165 ns/iter reference)
import jax
import jax.numpy as jnp
from jax import lax
from jax.experimental import pallas as pl
from jax.experimental.pallas import tpu as pltpu

N_DEV = 8
B = 512
D = 256
HS = 512
BP = 1024

PLANE_PEERS = (1, 3, 2)


def kernel(x, Win0, Wout0, Win1, Wout1, Win2, Wout2):
    def body(x_ref, win0_ref, wout0_ref, win1_ref, wout1_ref,
             win2_ref, wout2_ref, out_ref,
             winbuf, woutbuf, swin, swout, xin, psend, precv, totbuf,
             wsend_sems, wrecv_sems, msend_sems, mrecv_sems):
        me = lax.axis_index("i")
        myslot = lax.rem(me, 4)
        myrow = me // 4
        partner = jnp.bitwise_xor(me, 4)

        pending = []

        def send(src, dst, send_sem, recv_sem, dest):
            rdma = pltpu.make_async_remote_copy(
                src_ref=src, dst_ref=dst, send_sem=send_sem,
                recv_sem=recv_sem,
                device_id=(dest,), device_id_type=pl.DeviceIdType.MESH,
            )
            rdma.start()
            pending.append(rdma)

        def wait_recv(dst, recv_sem):
            rdma = pltpu.make_async_remote_copy(
                src_ref=dst, dst_ref=dst, send_sem=msend_sems.at[4],
                recv_sem=recv_sem,
                device_id=(me,), device_id_type=pl.DeviceIdType.MESH,
            )
            rdma.wait_recv()

        win_refs = [win0_ref, win1_ref, win2_ref]
        wout_refs = [wout0_ref, wout1_ref, wout2_ref]

        for l in range(3):
            swin[l] = win_refs[l][...].astype(jnp.bfloat16)
            swout[l] = wout_refs[l][...].astype(jnp.bfloat16)
        for l in range(3):
            for t in PLANE_PEERS:
                dest = jnp.bitwise_xor(me, t)
                dslot = lax.rem(dest, 4)
                send(swin.at[l], winbuf.at[l, myslot],
                     wsend_sems.at[l, 0, dslot], wrecv_sems.at[l, 0, myslot],
                     dest)
                send(swout.at[l], woutbuf.at[l, myslot],
                     wsend_sems.at[l, 1, dslot], wrecv_sems.at[l, 1, myslot],
                     dest)

        xin[0, pl.ds(myrow * B, B), :] = x_ref[...].astype(jnp.bfloat16)
        send(xin.at[0, pl.ds(myrow * B, B), :],
             xin.at[0, pl.ds(myrow * B, B), :],
             msend_sems.at[0], mrecv_sems.at[0], partner)
        wait_recv(xin.at[0, pl.ds((1 - myrow) * B, B), :], mrecv_sems.at[0])

        for l in range(3):
            X = xin[l]

            def contrib(win_s, wout_s):
                h = jnp.maximum(
                    jnp.dot(X, win_s, preferred_element_type=jnp.float32),
                    0.0,
                ).astype(jnp.bfloat16)
                return jnp.dot(h, wout_s, preferred_element_type=jnp.float32)

            acc = contrib(swin[l], swout[l])
            for t in PLANE_PEERS:
                s = lax.rem(jnp.bitwise_xor(me, t), 4)
                wait_recv(winbuf.at[l, s], wrecv_sems.at[l, 0, s])
                wait_recv(woutbuf.at[l, s], wrecv_sems.at[l, 1, s])
                acc = acc + contrib(winbuf[l, s], woutbuf[l, s])

            psend[l] = acc.astype(jnp.bfloat16)
            send(psend.at[l], precv.at[l],
                 msend_sems.at[1 + l], mrecv_sems.at[1 + l], partner)
            wait_recv(precv.at[l], mrecv_sems.at[1 + l])
            tot = acc + precv[l].astype(jnp.float32)

            if l < 2:
                xin[l + 1] = tot.astype(jnp.bfloat16)
            else:
                totbuf[...] = tot
                out_ref[...] = totbuf[pl.ds(myrow * B, B), :]

        for rdma in pending:
            rdma.wait_send()

    return pl.pallas_call(
        body,
        out_shape=jax.ShapeDtypeStruct((B, D), jnp.float32),
        in_specs=[pl.BlockSpec(memory_space=pltpu.VMEM)] * 7,
        out_specs=pl.BlockSpec(memory_space=pltpu.VMEM),
        scratch_shapes=[
            pltpu.VMEM((3, 4, D, HS), jnp.bfloat16),
            pltpu.VMEM((3, 4, HS, D), jnp.bfloat16),
            pltpu.VMEM((3, D, HS), jnp.bfloat16),
            pltpu.VMEM((3, HS, D), jnp.bfloat16),
            pltpu.VMEM((3, BP, D), jnp.bfloat16),
            pltpu.VMEM((3, BP, D), jnp.bfloat16),
            pltpu.VMEM((3, BP, D), jnp.bfloat16),
            pltpu.VMEM((BP, D), jnp.float32),
            pltpu.SemaphoreType.DMA((3, 2, 4)),
            pltpu.SemaphoreType.DMA((3, 2, 4)),
            pltpu.SemaphoreType.DMA((5,)),
            pltpu.SemaphoreType.DMA((4,)),
        ],
    )(x, Win0, Wout0, Win1, Wout1, Win2, Wout2)


# device time: 60451 ns/iter; 1.0118x vs baseline; 1.0118x over previous
import jax
import jax.numpy as jnp
from jax import lax
from jax.experimental import pallas as pl
from jax.experimental.pallas import tpu as pltpu

N_DEV = 8
B = 512
D = 256
HS = 512
BP = 1024

PLANE_PEERS = (1, 3, 2)


def kernel(x, Win0, Wout0, Win1, Wout1, Win2, Wout2):
    def body(x_ref, win0_ref, wout0_ref, win1_ref, wout1_ref,
             win2_ref, wout2_ref, out_ref,
             winbuf, woutbuf, swin, swout, xin, psend, precv,
             wsend_sems, wrecv_sems, msend_sems, mrecv_sems):
        me = lax.axis_index("i")
        myslot = lax.rem(me, 4)
        myrow = me // 4
        partner = jnp.bitwise_xor(me, 4)

        pending = []

        def send(src, dst, send_sem, recv_sem, dest):
            rdma = pltpu.make_async_remote_copy(
                src_ref=src, dst_ref=dst, send_sem=send_sem,
                recv_sem=recv_sem,
                device_id=(dest,), device_id_type=pl.DeviceIdType.MESH,
            )
            rdma.start()
            pending.append(rdma)

        def wait_recv(dst, recv_sem):
            rdma = pltpu.make_async_remote_copy(
                src_ref=dst, dst_ref=dst, send_sem=msend_sems.at[7],
                recv_sem=recv_sem,
                device_id=(me,), device_id_type=pl.DeviceIdType.MESH,
            )
            rdma.wait_recv()

        win_refs = [win0_ref, win1_ref, win2_ref]
        wout_refs = [wout0_ref, wout1_ref, wout2_ref]

        xin[0, pl.ds(myrow * B, B), :] = x_ref[...].astype(jnp.bfloat16)
        send(xin.at[0, pl.ds(myrow * B, B), :],
             xin.at[0, pl.ds(myrow * B, B), :],
             msend_sems.at[0], mrecv_sems.at[0], partner)

        for l in range(3):
            swin[l] = win_refs[l][...].astype(jnp.bfloat16)
            swout[l] = wout_refs[l][...].astype(jnp.bfloat16)
            for t in PLANE_PEERS:
                dest = jnp.bitwise_xor(me, t)
                dslot = lax.rem(dest, 4)
                send(swin.at[l], winbuf.at[l, myslot],
                     wsend_sems.at[l, 0, dslot], wrecv_sems.at[l, 0, myslot],
                     dest)
                send(swout.at[l], woutbuf.at[l, myslot],
                     wsend_sems.at[l, 1, dslot], wrecv_sems.at[l, 1, myslot],
                     dest)

        wait_recv(xin.at[0, pl.ds((1 - myrow) * B, B), :], mrecv_sems.at[0])

        for l in range(3):
            X = xin[l]
            X0, X1 = X[:B], X[B:]

            def contrib(xh, win_s, wout_s):
                h = jnp.maximum(
                    jnp.dot(xh, win_s, preferred_element_type=jnp.float32),
                    0.0,
                ).astype(jnp.bfloat16)
                return jnp.dot(h, wout_s, preferred_element_type=jnp.float32)

            acc0 = contrib(X0, swin[l], swout[l])
            for t in PLANE_PEERS:
                s = lax.rem(jnp.bitwise_xor(me, t), 4)
                wait_recv(winbuf.at[l, s], wrecv_sems.at[l, 0, s])
                wait_recv(woutbuf.at[l, s], wrecv_sems.at[l, 1, s])
                acc0 = acc0 + contrib(X0, winbuf[l, s], woutbuf[l, s])
            psend[l, :B] = acc0.astype(jnp.bfloat16)
            send(psend.at[l, pl.ds(0, B), :], precv.at[l, pl.ds(0, B), :],
                 msend_sems.at[1 + 2 * l], mrecv_sems.at[1 + 2 * l], partner)

            acc1 = contrib(X1, swin[l], swout[l])
            for t in PLANE_PEERS:
                s = lax.rem(jnp.bitwise_xor(me, t), 4)
                acc1 = acc1 + contrib(X1, winbuf[l, s], woutbuf[l, s])
            psend[l, B:] = acc1.astype(jnp.bfloat16)
            send(psend.at[l, pl.ds(B, B), :], precv.at[l, pl.ds(B, B), :],
                 msend_sems.at[2 + 2 * l], mrecv_sems.at[2 + 2 * l], partner)

            if l < 2:
                wait_recv(precv.at[l, pl.ds(0, B), :], mrecv_sems.at[1 + 2 * l])
                tot0 = acc0 + precv[l, :B].astype(jnp.float32)
                xin[l + 1, :B] = tot0.astype(jnp.bfloat16)
                wait_recv(precv.at[l, pl.ds(B, B), :], mrecv_sems.at[2 + 2 * l])
                tot1 = acc1 + precv[l, B:].astype(jnp.float32)
                xin[l + 1, B:] = tot1.astype(jnp.bfloat16)
            else:
                wait_recv(precv.at[l, pl.ds(0, B), :], mrecv_sems.at[1 + 2 * l])
                wait_recv(precv.at[l, pl.ds(B, B), :], mrecv_sems.at[2 + 2 * l])
                accmy = jnp.where(myrow == 0, acc0, acc1)
                out_ref[...] = accmy + precv[
                    l, pl.ds(myrow * B, B), :
                ].astype(jnp.float32)

        for rdma in pending:
            rdma.wait_send()

    return pl.pallas_call(
        body,
        out_shape=jax.ShapeDtypeStruct((B, D), jnp.float32),
        in_specs=[pl.BlockSpec(memory_space=pltpu.VMEM)] * 7,
        out_specs=pl.BlockSpec(memory_space=pltpu.VMEM),
        scratch_shapes=[
            pltpu.VMEM((3, 4, D, HS), jnp.bfloat16),
            pltpu.VMEM((3, 4, HS, D), jnp.bfloat16),
            pltpu.VMEM((3, D, HS), jnp.bfloat16),
            pltpu.VMEM((3, HS, D), jnp.bfloat16),
            pltpu.VMEM((3, BP, D), jnp.bfloat16),
            pltpu.VMEM((3, BP, D), jnp.bfloat16),
            pltpu.VMEM((3, BP, D), jnp.bfloat16),
            pltpu.SemaphoreType.DMA((3, 2, 4)),
            pltpu.SemaphoreType.DMA((3, 2, 4)),
            pltpu.SemaphoreType.DMA((8,)),
            pltpu.SemaphoreType.DMA((8,)),
        ],
    )(x, Win0, Wout0, Win1, Wout1, Win2, Wout2)


# device time: 55346 ns/iter; 1.1051x vs baseline; 1.0922x over previous
import jax
import jax.numpy as jnp
from jax import lax
from jax.experimental import pallas as pl
from jax.experimental.pallas import tpu as pltpu

N_DEV = 8
B = 512
D = 256
HS = 512
BP = 1024

PLANE_PEERS = (1, 3, 2)


def kernel(x, Win0, Wout0, Win1, Wout1, Win2, Wout2):
    def body(x_ref, win0_ref, wout0_ref, win1_ref, wout1_ref,
             win2_ref, wout2_ref, out_ref,
             winbuf, woutbuf, swin, swout, xin, psend, precv,
             wsend_sems, wrecv_sems, msend_sems, mrecv_sems):
        me = lax.axis_index("i")
        myslot = lax.rem(me, 4)
        myrow = me // 4
        partner = jnp.bitwise_xor(me, 4)

        pending = []

        barrier_sem = pltpu.get_barrier_semaphore()
        for t in (1, 3, 2, 4):
            pl.semaphore_signal(
                barrier_sem, inc=1,
                device_id=(jnp.bitwise_xor(me, t),),
                device_id_type=pl.DeviceIdType.MESH,
            )
        pl.semaphore_wait(barrier_sem, 4)

        def send(src, dst, send_sem, recv_sem, dest):
            rdma = pltpu.make_async_remote_copy(
                src_ref=src, dst_ref=dst, send_sem=send_sem,
                recv_sem=recv_sem,
                device_id=(dest,), device_id_type=pl.DeviceIdType.MESH,
            )
            rdma.start()
            pending.append(rdma)

        def wait_recv(dst, recv_sem):
            rdma = pltpu.make_async_remote_copy(
                src_ref=dst, dst_ref=dst, send_sem=msend_sems.at[7],
                recv_sem=recv_sem,
                device_id=(me,), device_id_type=pl.DeviceIdType.MESH,
            )
            rdma.wait_recv()

        win_refs = [win0_ref, win1_ref, win2_ref]
        wout_refs = [wout0_ref, wout1_ref, wout2_ref]

        xin[0, pl.ds(myrow * B, B), :] = x_ref[...].astype(jnp.bfloat16)
        send(xin.at[0, pl.ds(myrow * B, B), :],
             xin.at[0, pl.ds(myrow * B, B), :],
             msend_sems.at[0], mrecv_sems.at[0], partner)

        for l in range(3):
            swin[l] = win_refs[l][...].astype(jnp.bfloat16)
            swout[l] = wout_refs[l][...].astype(jnp.bfloat16)
            for t in PLANE_PEERS:
                dest = jnp.bitwise_xor(me, t)
                dslot = lax.rem(dest, 4)
                send(swin.at[l], winbuf.at[l, myslot],
                     wsend_sems.at[l, 0, dslot], wrecv_sems.at[l, 0, myslot],
                     dest)
                send(swout.at[l], woutbuf.at[l, myslot],
                     wsend_sems.at[l, 1, dslot], wrecv_sems.at[l, 1, myslot],
                     dest)

        wait_recv(xin.at[0, pl.ds((1 - myrow) * B, B), :], mrecv_sems.at[0])

        for l in range(3):
            X = xin[l]
            X0, X1 = X[:B], X[B:]

            def contrib(xh, win_s, wout_s):
                h = jnp.maximum(
                    jnp.dot(xh, win_s, preferred_element_type=jnp.float32),
                    0.0,
                ).astype(jnp.bfloat16)
                return jnp.dot(h, wout_s, preferred_element_type=jnp.float32)

            acc0 = contrib(X0, swin[l], swout[l])
            for t in PLANE_PEERS:
                s = lax.rem(jnp.bitwise_xor(me, t), 4)
                wait_recv(winbuf.at[l, s], wrecv_sems.at[l, 0, s])
                wait_recv(woutbuf.at[l, s], wrecv_sems.at[l, 1, s])
                acc0 = acc0 + contrib(X0, winbuf[l, s], woutbuf[l, s])
            psend[l, :B] = acc0.astype(jnp.bfloat16)
            if l < 2:
                send(psend.at[l, pl.ds(0, B), :], precv.at[l, pl.ds(0, B), :],
                     msend_sems.at[1 + 2 * l], mrecv_sems.at[1 + 2 * l],
                     partner)

            acc1 = contrib(X1, swin[l], swout[l])
            for t in PLANE_PEERS:
                s = lax.rem(jnp.bitwise_xor(me, t), 4)
                acc1 = acc1 + contrib(X1, winbuf[l, s], woutbuf[l, s])
            psend[l, B:] = acc1.astype(jnp.bfloat16)
            if l < 2:
                send(psend.at[l, pl.ds(B, B), :], precv.at[l, pl.ds(B, B), :],
                     msend_sems.at[2 + 2 * l], mrecv_sems.at[2 + 2 * l],
                     partner)
                wait_recv(precv.at[l, pl.ds(0, B), :], mrecv_sems.at[1 + 2 * l])
                tot0 = acc0 + precv[l, :B].astype(jnp.float32)
                xin[l + 1, :B] = tot0.astype(jnp.bfloat16)
                wait_recv(precv.at[l, pl.ds(B, B), :], mrecv_sems.at[2 + 2 * l])
                tot1 = acc1 + precv[l, B:].astype(jnp.float32)
                xin[l + 1, B:] = tot1.astype(jnp.bfloat16)
            else:
                send(psend.at[l, pl.ds((1 - myrow) * B, B), :],
                     precv.at[l, pl.ds((1 - myrow) * B, B), :],
                     msend_sems.at[2 + 2 * l], mrecv_sems.at[2 + 2 * l],
                     partner)
                wait_recv(precv.at[l, pl.ds(myrow * B, B), :],
                          mrecv_sems.at[2 + 2 * l])
                accmy = jnp.where(myrow == 0, acc0, acc1)
                out_ref[...] = accmy + precv[
                    l, pl.ds(myrow * B, B), :
                ].astype(jnp.float32)

        for rdma in pending:
            rdma.wait_send()

    return pl.pallas_call(
        body,
        out_shape=jax.ShapeDtypeStruct((B, D), jnp.float32),
        in_specs=[pl.BlockSpec(memory_space=pltpu.VMEM)] * 7,
        out_specs=pl.BlockSpec(memory_space=pltpu.VMEM),
        scratch_shapes=[
            pltpu.VMEM((3, 4, D, HS), jnp.bfloat16),
            pltpu.VMEM((3, 4, HS, D), jnp.bfloat16),
            pltpu.VMEM((3, D, HS), jnp.bfloat16),
            pltpu.VMEM((3, HS, D), jnp.bfloat16),
            pltpu.VMEM((3, BP, D), jnp.bfloat16),
            pltpu.VMEM((3, BP, D), jnp.bfloat16),
            pltpu.VMEM((3, BP, D), jnp.bfloat16),
            pltpu.SemaphoreType.DMA((3, 2, 4)),
            pltpu.SemaphoreType.DMA((3, 2, 4)),
            pltpu.SemaphoreType.DMA((8,)),
            pltpu.SemaphoreType.DMA((8,)),
        ],
        compiler_params=pltpu.CompilerParams(collective_id=0),
    )(x, Win0, Wout0, Win1, Wout1, Win2, Wout2)
